# Initial kernel scaffold; baseline (speedup 1.0000x reference)
import jax
import jax.numpy as jnp
from jax.experimental import pallas as pl


def kernel(points, inverse_indices, W1, b1, g1, be1, W2, b2, g2, be2):
    raise NotImplementedError("write your pallas kernel here")


# speedup vs baseline: 1.6331x; 1.6331x over previous
import jax
import jax.numpy as jnp
from jax import lax
from jax.experimental import pallas as pl
from jax.experimental.pallas import tpu as pltpu
from jax.experimental.pallas import tpu_sc as plsc

N_POINTS = 1600000
N_SEG = 100000
F1 = 9
F2 = 32
EPS = 1e-5

BLK = 2000
GRID = N_POINTS // BLK
CH = 128
N_PAD = N_POINTS + CH
N_TILES = 32
SEG_T = 3128
N_SEG_PAD = N_TILES * SEG_T


def _stats1_body(x_ref, w1t_ref, b1_ref, out_ref):
    i = pl.program_id(0)
    h = jnp.dot(x_ref[...], w1t_ref[...], preferred_element_type=jnp.float32)
    h = h + b1_ref[...]
    s = jnp.sum(h, axis=0, keepdims=True)
    ss = jnp.sum(h * h, axis=0, keepdims=True)
    blk = jnp.concatenate([s, ss], axis=0)

    @pl.when(i == 0)
    def _():
        out_ref[...] = blk

    @pl.when(i != 0)
    def _():
        out_ref[...] = out_ref[...] + blk


def _fwd_body(x_ref, w1t_ref, b1_ref, sc1_ref, sh1_ref, w2t_ref, b2_ref,
              h2_ref, st_ref):
    i = pl.program_id(0)
    h1 = jnp.dot(x_ref[...], w1t_ref[...], preferred_element_type=jnp.float32)
    h1 = h1 + b1_ref[...]
    f1 = jnp.maximum(h1 * sc1_ref[...] + sh1_ref[...], 0.0)
    h2 = jnp.dot(f1, w2t_ref[...], preferred_element_type=jnp.float32)
    h2 = h2 + b2_ref[...]
    h2_ref[...] = h2
    s = jnp.sum(h2, axis=0, keepdims=True)
    ss = jnp.sum(h2 * h2, axis=0, keepdims=True)
    blk = jnp.concatenate([s, ss], axis=0)

    @pl.when(i == 0)
    def _():
        st_ref[...] = blk

    @pl.when(i != 0)
    def _():
        st_ref[...] = st_ref[...] + blk


def _segmax_body(h2_hbm, idx_hbm, splits_hbm, aff_hbm, out_hbm,
                 splits_v, aff_v, idxbuf, rowbuf, slab):
    cid = lax.axis_index("c")
    sid = lax.axis_index("s")
    wid = sid * 2 + cid
    pltpu.sync_copy(splits_hbm, splits_v)
    pltpu.sync_copy(aff_hbm, aff_v)
    seg_lo = pl.multiple_of(wid * SEG_T, 8)
    rvec = splits_v[pl.ds(wid, 16)]
    r0 = rvec[0]
    r1 = rvec[1]
    r0a = jnp.bitwise_and(r0, jnp.int32(-8))
    nchunks = (r1 - r0a + (CH - 1)) // CH

    ninf = jnp.full((16,), -jnp.inf, dtype=jnp.float32)

    def init_body(s, carry):
        slab[pl.ds(s * F2, 16)] = ninf
        slab[pl.ds(s * F2 + 16, 16)] = ninf
        return carry

    lax.fori_loop(0, SEG_T, init_body, 0)

    def do_row(local, r):
        @pl.when(jnp.logical_and(local >= 0, local < SEG_T))
        def _():
            o = local * F2
            slab[pl.ds(o, 16)] = jnp.maximum(slab[pl.ds(o, 16)],
                                             rowbuf[r, 0:16])
            slab[pl.ds(o + 16, 16)] = jnp.maximum(slab[pl.ds(o + 16, 16)],
                                                  rowbuf[r, 16:32])

    def group_body(g, carry):
        iv = idxbuf[pl.ds(g * 16, 16)]
        for j in range(16):
            do_row(iv[j] - seg_lo, g * 16 + j)
        return carry

    def chunk_body(c, carry):
        base = pl.multiple_of(r0a + c * CH, 8)
        pltpu.sync_copy(idx_hbm.at[pl.ds(base, CH)], idxbuf)
        pltpu.sync_copy(h2_hbm.at[pl.ds(base, CH), :], rowbuf)
        lax.fori_loop(0, CH // 16, group_body, 0)
        return carry

    lax.fori_loop(0, nchunks, chunk_body, 0)

    sca = aff_v[pl.ds(0, 16)]
    scb = aff_v[pl.ds(16, 16)]
    sha = aff_v[pl.ds(32, 16)]
    shb = aff_v[pl.ds(48, 16)]
    zero = jnp.zeros((16,), dtype=jnp.float32)

    def fin_body(s, carry):
        o = s * F2
        slab[pl.ds(o, 16)] = jnp.maximum(slab[pl.ds(o, 16)] * sca + sha, zero)
        slab[pl.ds(o + 16, 16)] = jnp.maximum(
            slab[pl.ds(o + 16, 16)] * scb + shb, zero)
        return carry

    lax.fori_loop(0, SEG_T, fin_body, 0)
    out_lo = pl.multiple_of(wid * (SEG_T * F2), 8)
    pltpu.sync_copy(slab, out_hbm.at[pl.ds(out_lo, SEG_T * F2)])


def kernel(points, inverse_indices, W1, b1, g1, be1, W2, b2, g2, be2):
    idx = inverse_indices.astype(jnp.int32)
    w1t = W1.T
    w2t = W2.T
    b1r = b1.reshape(1, F2)

    st1 = pl.pallas_call(
        _stats1_body,
        grid=(GRID,),
        in_specs=[
            pl.BlockSpec((BLK, F1), lambda i: (i, 0)),
            pl.BlockSpec((F1, F2), lambda i: (0, 0)),
            pl.BlockSpec((1, F2), lambda i: (0, 0)),
        ],
        out_specs=pl.BlockSpec((2, F2), lambda i: (0, 0)),
        out_shape=jax.ShapeDtypeStruct((2, F2), jnp.float32),
        compiler_params=pltpu.CompilerParams(
            dimension_semantics=("arbitrary",)),
    )(points, w1t, b1r)

    inv_n = jnp.float32(1.0 / N_POINTS)
    mean1 = st1[0] * inv_n
    var1 = st1[1] * inv_n - mean1 * mean1
    sc1 = g1 / jnp.sqrt(var1 + EPS)
    sh1 = be1 - mean1 * sc1

    h2p, st2 = pl.pallas_call(
        _fwd_body,
        grid=(GRID,),
        in_specs=[
            pl.BlockSpec((BLK, F1), lambda i: (i, 0)),
            pl.BlockSpec((F1, F2), lambda i: (0, 0)),
            pl.BlockSpec((1, F2), lambda i: (0, 0)),
            pl.BlockSpec((1, F2), lambda i: (0, 0)),
            pl.BlockSpec((1, F2), lambda i: (0, 0)),
            pl.BlockSpec((F2, F2), lambda i: (0, 0)),
            pl.BlockSpec((1, F2), lambda i: (0, 0)),
        ],
        out_specs=[
            pl.BlockSpec((BLK, F2), lambda i: (i, 0)),
            pl.BlockSpec((2, F2), lambda i: (0, 0)),
        ],
        out_shape=[
            jax.ShapeDtypeStruct((N_PAD, F2), jnp.float32),
            jax.ShapeDtypeStruct((2, F2), jnp.float32),
        ],
        compiler_params=pltpu.CompilerParams(
            dimension_semantics=("arbitrary",)),
    )(points, w1t, b1r, sc1.reshape(1, F2), sh1.reshape(1, F2), w2t,
      b2.reshape(1, F2))

    mean2 = st2[0] * inv_n
    var2 = st2[1] * inv_n - mean2 * mean2
    sc2 = g2 / jnp.sqrt(var2 + EPS)
    sh2 = be2 - mean2 * sc2
    aff = jnp.concatenate([sc2, sh2]).astype(jnp.float32)

    idx_pad = jnp.concatenate(
        [idx, jnp.full((CH,), jnp.int32(2**31 - 2))])
    bounds = jnp.arange(0, N_SEG_PAD + 1, SEG_T, dtype=jnp.int32)
    splits = jnp.searchsorted(idx, bounds).astype(jnp.int32)
    splits = jnp.concatenate([splits, jnp.zeros((15,), jnp.int32)])

    mesh = plsc.VectorSubcoreMesh(core_axis_name="c", subcore_axis_name="s")
    out = pl.kernel(
        _segmax_body,
        mesh=mesh,
        out_type=jax.ShapeDtypeStruct((N_SEG_PAD * F2,), jnp.float32),
        scratch_types=[
            pltpu.VMEM((48,), jnp.int32),
            pltpu.VMEM((2 * F2,), jnp.float32),
            pltpu.VMEM((CH,), jnp.int32),
            pltpu.VMEM((CH, F2), jnp.float32),
            pltpu.VMEM((SEG_T * F2,), jnp.float32),
        ],
    )(h2p, idx_pad, splits, aff)
    return out.reshape(N_SEG_PAD, F2)[:N_SEG]


# speedup vs baseline: 2.0146x; 1.2336x over previous
import jax
import jax.numpy as jnp
from jax import lax
from jax.experimental import pallas as pl
from jax.experimental.pallas import tpu as pltpu
from jax.experimental.pallas import tpu_sc as plsc

N_POINTS = 1600000
N_SEG = 100000
F1 = 9
F2 = 32
EPS = 1e-5

BLK = 2000
GRID = N_POINTS // BLK
CH = 112
N_PAD = N_POINTS + CH
N_TILES = 32
SEG_T = 3128
N_SEG_PAD = N_TILES * SEG_T


def _stats1_body(x_ref, w1t_ref, b1_ref, out_ref):
    i = pl.program_id(0)
    h = jnp.dot(x_ref[...], w1t_ref[...], preferred_element_type=jnp.float32)
    h = h + b1_ref[...]
    s = jnp.sum(h, axis=0, keepdims=True)
    ss = jnp.sum(h * h, axis=0, keepdims=True)
    blk = jnp.concatenate([s, ss], axis=0)

    @pl.when(i == 0)
    def _():
        out_ref[...] = blk

    @pl.when(i != 0)
    def _():
        out_ref[...] = out_ref[...] + blk


def _fwd_body(x_ref, w1t_ref, b1_ref, sc1_ref, sh1_ref, w2t_ref, b2_ref,
              h2_ref, st_ref):
    i = pl.program_id(0)
    h1 = jnp.dot(x_ref[...], w1t_ref[...], preferred_element_type=jnp.float32)
    h1 = h1 + b1_ref[...]
    f1 = jnp.maximum(h1 * sc1_ref[...] + sh1_ref[...], 0.0)
    h2 = jnp.dot(f1, w2t_ref[...], preferred_element_type=jnp.float32)
    h2 = h2 + b2_ref[...]
    h2_ref[...] = h2
    s = jnp.sum(h2, axis=0, keepdims=True)
    ss = jnp.sum(h2 * h2, axis=0, keepdims=True)
    blk = jnp.concatenate([s, ss], axis=0)

    @pl.when(i == 0)
    def _():
        st_ref[...] = blk

    @pl.when(i != 0)
    def _():
        st_ref[...] = st_ref[...] + blk


def _segmax_body(h2_hbm, idx_hbm, splits_hbm, aff_hbm, out_hbm,
                 splits_v, aff_v, idxbuf0, idxbuf1, rowbuf0, rowbuf1,
                 slab, sem0, sem1):
    cid = lax.axis_index("c")
    sid = lax.axis_index("s")
    wid = sid * 2 + cid
    pltpu.sync_copy(splits_hbm, splits_v)
    pltpu.sync_copy(aff_hbm, aff_v)
    seg_lo = pl.multiple_of(wid * SEG_T, 8)
    rvec = splits_v[pl.ds(wid, 16)]
    r0 = rvec[0]
    r1 = rvec[1]
    r0a = jnp.bitwise_and(r0, jnp.int32(-8))
    nchunks = (r1 - r0a + (CH - 1)) // CH

    def start(c, ib, rb, sem):
        base = pl.multiple_of(r0a + c * CH, 8)
        pltpu.async_copy(idx_hbm.at[pl.ds(base, CH)], ib, sem)
        pltpu.async_copy(h2_hbm.at[pl.ds(base, CH), :], rb, sem)

    def drain(ib, rb, sem):
        pltpu.make_async_copy(idx_hbm.at[pl.ds(0, CH)], ib, sem).wait()
        pltpu.make_async_copy(h2_hbm.at[pl.ds(0, CH), :], rb, sem).wait()

    @pl.when(nchunks > 0)
    def _():
        start(0, idxbuf0, rowbuf0, sem0)

    @pl.when(nchunks > 1)
    def _():
        start(1, idxbuf1, rowbuf1, sem1)

    ninf = jnp.full((16,), -jnp.inf, dtype=jnp.float32)

    def init_body(s, carry):
        slab[pl.ds(s * F2, 16)] = ninf
        slab[pl.ds(s * F2 + 16, 16)] = ninf
        return carry

    lax.fori_loop(0, SEG_T, init_body, 0)

    def do_row(local, rb, r):
        @pl.when(jnp.logical_and(local >= 0, local < SEG_T))
        def _():
            o = local * F2
            slab[pl.ds(o, 16)] = jnp.maximum(slab[pl.ds(o, 16)],
                                             rb[r, 0:16])
            slab[pl.ds(o + 16, 16)] = jnp.maximum(slab[pl.ds(o + 16, 16)],
                                                  rb[r, 16:32])

    def process(ib, rb):
        def group_body(g, carry):
            iv = ib[pl.ds(g * 16, 16)]
            for j in range(16):
                do_row(iv[j] - seg_lo, rb, g * 16 + j)
            return carry
        lax.fori_loop(0, CH // 16, group_body, 0)

    def pair_body(k, carry):
        c0 = 2 * k
        c1 = c0 + 1

        @pl.when(c0 < nchunks)
        def _():
            drain(idxbuf0, rowbuf0, sem0)
            process(idxbuf0, rowbuf0)

            @pl.when(c0 + 2 < nchunks)
            def _():
                start(c0 + 2, idxbuf0, rowbuf0, sem0)

        @pl.when(c1 < nchunks)
        def _():
            drain(idxbuf1, rowbuf1, sem1)
            process(idxbuf1, rowbuf1)

            @pl.when(c1 + 2 < nchunks)
            def _():
                start(c1 + 2, idxbuf1, rowbuf1, sem1)

        return carry

    lax.fori_loop(0, (nchunks + 1) // 2, pair_body, 0)

    sca = aff_v[pl.ds(0, 16)]
    scb = aff_v[pl.ds(16, 16)]
    sha = aff_v[pl.ds(32, 16)]
    shb = aff_v[pl.ds(48, 16)]
    zero = jnp.zeros((16,), dtype=jnp.float32)

    def fin_body(s, carry):
        o = s * F2
        slab[pl.ds(o, 16)] = jnp.maximum(slab[pl.ds(o, 16)] * sca + sha, zero)
        slab[pl.ds(o + 16, 16)] = jnp.maximum(
            slab[pl.ds(o + 16, 16)] * scb + shb, zero)
        return carry

    lax.fori_loop(0, SEG_T, fin_body, 0)
    out_lo = pl.multiple_of(wid * (SEG_T * F2), 8)
    pltpu.sync_copy(slab, out_hbm.at[pl.ds(out_lo, SEG_T * F2)])


def kernel(points, inverse_indices, W1, b1, g1, be1, W2, b2, g2, be2):
    idx = inverse_indices.astype(jnp.int32)
    w1t = W1.T
    w2t = W2.T
    b1r = b1.reshape(1, F2)

    st1 = pl.pallas_call(
        _stats1_body,
        grid=(GRID,),
        in_specs=[
            pl.BlockSpec((BLK, F1), lambda i: (i, 0)),
            pl.BlockSpec((F1, F2), lambda i: (0, 0)),
            pl.BlockSpec((1, F2), lambda i: (0, 0)),
        ],
        out_specs=pl.BlockSpec((2, F2), lambda i: (0, 0)),
        out_shape=jax.ShapeDtypeStruct((2, F2), jnp.float32),
        compiler_params=pltpu.CompilerParams(
            dimension_semantics=("arbitrary",)),
    )(points, w1t, b1r)

    inv_n = jnp.float32(1.0 / N_POINTS)
    mean1 = st1[0] * inv_n
    var1 = st1[1] * inv_n - mean1 * mean1
    sc1 = g1 / jnp.sqrt(var1 + EPS)
    sh1 = be1 - mean1 * sc1

    h2p, st2 = pl.pallas_call(
        _fwd_body,
        grid=(GRID,),
        in_specs=[
            pl.BlockSpec((BLK, F1), lambda i: (i, 0)),
            pl.BlockSpec((F1, F2), lambda i: (0, 0)),
            pl.BlockSpec((1, F2), lambda i: (0, 0)),
            pl.BlockSpec((1, F2), lambda i: (0, 0)),
            pl.BlockSpec((1, F2), lambda i: (0, 0)),
            pl.BlockSpec((F2, F2), lambda i: (0, 0)),
            pl.BlockSpec((1, F2), lambda i: (0, 0)),
        ],
        out_specs=[
            pl.BlockSpec((BLK, F2), lambda i: (i, 0)),
            pl.BlockSpec((2, F2), lambda i: (0, 0)),
        ],
        out_shape=[
            jax.ShapeDtypeStruct((N_PAD, F2), jnp.float32),
            jax.ShapeDtypeStruct((2, F2), jnp.float32),
        ],
        compiler_params=pltpu.CompilerParams(
            dimension_semantics=("arbitrary",)),
    )(points, w1t, b1r, sc1.reshape(1, F2), sh1.reshape(1, F2), w2t,
      b2.reshape(1, F2))

    mean2 = st2[0] * inv_n
    var2 = st2[1] * inv_n - mean2 * mean2
    sc2 = g2 / jnp.sqrt(var2 + EPS)
    sh2 = be2 - mean2 * sc2
    aff = jnp.concatenate([sc2, sh2]).astype(jnp.float32)

    idx_pad = jnp.concatenate(
        [idx, jnp.full((CH,), jnp.int32(2**31 - 2))])
    bounds = jnp.arange(0, N_SEG_PAD + 1, SEG_T, dtype=jnp.int32)
    splits = jnp.searchsorted(idx, bounds).astype(jnp.int32)
    splits = jnp.concatenate([splits, jnp.zeros((15,), jnp.int32)])

    mesh = plsc.VectorSubcoreMesh(core_axis_name="c", subcore_axis_name="s")
    out = pl.kernel(
        _segmax_body,
        mesh=mesh,
        out_type=jax.ShapeDtypeStruct((N_SEG_PAD * F2,), jnp.float32),
        scratch_types=[
            pltpu.VMEM((48,), jnp.int32),
            pltpu.VMEM((2 * F2,), jnp.float32),
            pltpu.VMEM((CH,), jnp.int32),
            pltpu.VMEM((CH,), jnp.int32),
            pltpu.VMEM((CH, F2), jnp.float32),
            pltpu.VMEM((CH, F2), jnp.float32),
            pltpu.VMEM((SEG_T * F2,), jnp.float32),
            pltpu.SemaphoreType.DMA,
            pltpu.SemaphoreType.DMA,
        ],
    )(h2p, idx_pad, splits, aff)
    return out.reshape(N_SEG_PAD, F2)[:N_SEG]


# speedup vs baseline: 2.4351x; 1.2087x over previous
import jax
import jax.numpy as jnp
from jax import lax
from jax.experimental import pallas as pl
from jax.experimental.pallas import tpu as pltpu
from jax.experimental.pallas import tpu_sc as plsc

N_POINTS = 1600000
N_SEG = 100000
F1 = 9
F2 = 32
EPS = 1e-5

BLK = 4000
GRID = N_POINTS // BLK
CH = 112
N_PAD = N_POINTS + CH
N_TILES = 32
SEG_T = 3128
N_SEG_PAD = N_TILES * SEG_T


def _stats1_body(x_ref, w1t_ref, b1_ref, out_ref):
    i = pl.program_id(0)
    h = jnp.dot(x_ref[...], w1t_ref[...], preferred_element_type=jnp.float32)
    h = h + b1_ref[...]
    s = jnp.sum(h, axis=0, keepdims=True)
    ss = jnp.sum(h * h, axis=0, keepdims=True)
    blk = jnp.concatenate([s, ss], axis=0)

    @pl.when(i == 0)
    def _():
        out_ref[...] = blk

    @pl.when(i != 0)
    def _():
        out_ref[...] = out_ref[...] + blk


def _fwd_body(x_ref, w1t_ref, b1_ref, sc1_ref, sh1_ref, w2t_ref, b2_ref,
              h2_ref, st_ref):
    i = pl.program_id(0)
    h1 = jnp.dot(x_ref[...], w1t_ref[...], preferred_element_type=jnp.float32)
    h1 = h1 + b1_ref[...]
    f1 = jnp.maximum(h1 * sc1_ref[...] + sh1_ref[...], 0.0)
    h2 = jnp.dot(f1, w2t_ref[...], preferred_element_type=jnp.float32)
    h2 = h2 + b2_ref[...]
    h2_ref[...] = h2
    s = jnp.sum(h2, axis=0, keepdims=True)
    ss = jnp.sum(h2 * h2, axis=0, keepdims=True)
    blk = jnp.concatenate([s, ss], axis=0)

    @pl.when(i == 0)
    def _():
        st_ref[...] = blk

    @pl.when(i != 0)
    def _():
        st_ref[...] = st_ref[...] + blk


def _segmax_body(h2_hbm, idx_hbm, splits_hbm, aff_hbm, out_hbm,
                 splits_v, aff_v, idxbuf0, idxbuf1, rowbuf0, rowbuf1,
                 slab, sem0, sem1):
    cid = lax.axis_index("c")
    sid = lax.axis_index("s")
    wid = sid * 2 + cid
    pltpu.sync_copy(splits_hbm, splits_v)
    pltpu.sync_copy(aff_hbm, aff_v)
    seg_lo = pl.multiple_of(wid * SEG_T, 8)
    rvec = splits_v[pl.ds(wid, 16)]
    r0 = rvec[0]
    r1 = rvec[1]
    r0a = jnp.bitwise_and(r0, jnp.int32(-8))
    nchunks = (r1 - r0a + (CH - 1)) // CH

    def start(c, ib, rb, sem):
        base = pl.multiple_of(r0a + c * CH, 8)
        pltpu.async_copy(idx_hbm.at[pl.ds(base, CH)], ib, sem)
        pltpu.async_copy(h2_hbm.at[pl.ds(base, CH), :], rb, sem)

    def drain(ib, rb, sem):
        pltpu.make_async_copy(idx_hbm.at[pl.ds(0, CH)], ib, sem).wait()
        pltpu.make_async_copy(h2_hbm.at[pl.ds(0, CH), :], rb, sem).wait()

    @pl.when(nchunks > 0)
    def _():
        start(0, idxbuf0, rowbuf0, sem0)

    @pl.when(nchunks > 1)
    def _():
        start(1, idxbuf1, rowbuf1, sem1)

    ninf = jnp.full((16,), -jnp.inf, dtype=jnp.float32)

    def init_body(s, carry):
        slab[pl.ds(s * F2, 16)] = ninf
        slab[pl.ds(s * F2 + 16, 16)] = ninf
        return carry

    lax.fori_loop(0, SEG_T, init_body, 0)

    def do_row(local, rb, r):
        @pl.when(jnp.logical_and(local >= 0, local < SEG_T))
        def _():
            o = local * F2
            slab[pl.ds(o, 16)] = jnp.maximum(slab[pl.ds(o, 16)],
                                             rb[r, 0:16])
            slab[pl.ds(o + 16, 16)] = jnp.maximum(slab[pl.ds(o + 16, 16)],
                                                  rb[r, 16:32])

    def process(ib, rb):
        def group_body(g, carry):
            iv = ib[pl.ds(g * 16, 16)]
            for j in range(16):
                do_row(iv[j] - seg_lo, rb, g * 16 + j)
            return carry
        lax.fori_loop(0, CH // 16, group_body, 0)

    def pair_body(k, carry):
        c0 = 2 * k
        c1 = c0 + 1

        @pl.when(c0 < nchunks)
        def _():
            drain(idxbuf0, rowbuf0, sem0)
            process(idxbuf0, rowbuf0)

            @pl.when(c0 + 2 < nchunks)
            def _():
                start(c0 + 2, idxbuf0, rowbuf0, sem0)

        @pl.when(c1 < nchunks)
        def _():
            drain(idxbuf1, rowbuf1, sem1)
            process(idxbuf1, rowbuf1)

            @pl.when(c1 + 2 < nchunks)
            def _():
                start(c1 + 2, idxbuf1, rowbuf1, sem1)

        return carry

    lax.fori_loop(0, (nchunks + 1) // 2, pair_body, 0)

    sca = aff_v[pl.ds(0, 16)]
    scb = aff_v[pl.ds(16, 16)]
    sha = aff_v[pl.ds(32, 16)]
    shb = aff_v[pl.ds(48, 16)]
    zero = jnp.zeros((16,), dtype=jnp.float32)

    def fin_body(s, carry):
        o = s * F2
        slab[pl.ds(o, 16)] = jnp.maximum(slab[pl.ds(o, 16)] * sca + sha, zero)
        slab[pl.ds(o + 16, 16)] = jnp.maximum(
            slab[pl.ds(o + 16, 16)] * scb + shb, zero)
        return carry

    lax.fori_loop(0, SEG_T, fin_body, 0)
    out_lo = pl.multiple_of(wid * (SEG_T * F2), 8)
    pltpu.sync_copy(slab, out_hbm.at[pl.ds(out_lo, SEG_T * F2)])


def kernel(points, inverse_indices, W1, b1, g1, be1, W2, b2, g2, be2):
    idx = inverse_indices.astype(jnp.int32)
    w1t = W1.T
    w2t = W2.T
    b1r = b1.reshape(1, F2)

    st1 = pl.pallas_call(
        _stats1_body,
        grid=(GRID,),
        in_specs=[
            pl.BlockSpec((BLK, F1), lambda i: (i, 0)),
            pl.BlockSpec((F1, F2), lambda i: (0, 0)),
            pl.BlockSpec((1, F2), lambda i: (0, 0)),
        ],
        out_specs=pl.BlockSpec((2, F2), lambda i: (0, 0)),
        out_shape=jax.ShapeDtypeStruct((2, F2), jnp.float32),
        compiler_params=pltpu.CompilerParams(
            dimension_semantics=("arbitrary",)),
    )(points, w1t, b1r)

    inv_n = jnp.float32(1.0 / N_POINTS)
    mean1 = st1[0] * inv_n
    var1 = st1[1] * inv_n - mean1 * mean1
    sc1 = g1 / jnp.sqrt(var1 + EPS)
    sh1 = be1 - mean1 * sc1

    h2p, st2 = pl.pallas_call(
        _fwd_body,
        grid=(GRID,),
        in_specs=[
            pl.BlockSpec((BLK, F1), lambda i: (i, 0)),
            pl.BlockSpec((F1, F2), lambda i: (0, 0)),
            pl.BlockSpec((1, F2), lambda i: (0, 0)),
            pl.BlockSpec((1, F2), lambda i: (0, 0)),
            pl.BlockSpec((1, F2), lambda i: (0, 0)),
            pl.BlockSpec((F2, F2), lambda i: (0, 0)),
            pl.BlockSpec((1, F2), lambda i: (0, 0)),
        ],
        out_specs=[
            pl.BlockSpec((BLK, F2), lambda i: (i, 0)),
            pl.BlockSpec((2, F2), lambda i: (0, 0)),
        ],
        out_shape=[
            jax.ShapeDtypeStruct((N_PAD, F2), jnp.float32),
            jax.ShapeDtypeStruct((2, F2), jnp.float32),
        ],
        compiler_params=pltpu.CompilerParams(
            dimension_semantics=("arbitrary",)),
    )(points, w1t, b1r, sc1.reshape(1, F2), sh1.reshape(1, F2), w2t,
      b2.reshape(1, F2))

    mean2 = st2[0] * inv_n
    var2 = st2[1] * inv_n - mean2 * mean2
    sc2 = g2 / jnp.sqrt(var2 + EPS)
    sh2 = be2 - mean2 * sc2
    aff = jnp.concatenate([sc2, sh2]).astype(jnp.float32)

    idx_pad = jnp.concatenate(
        [idx, jnp.full((CH,), jnp.int32(2**31 - 2))])
    bounds = jnp.arange(0, N_SEG_PAD + 1, SEG_T, dtype=jnp.int32)
    splits = jnp.searchsorted(idx, bounds).astype(jnp.int32)
    splits = jnp.concatenate([splits, jnp.zeros((15,), jnp.int32)])

    mesh = plsc.VectorSubcoreMesh(core_axis_name="c", subcore_axis_name="s")
    out = pl.kernel(
        _segmax_body,
        mesh=mesh,
        out_type=jax.ShapeDtypeStruct((N_SEG_PAD * F2,), jnp.float32),
        scratch_types=[
            pltpu.VMEM((48,), jnp.int32),
            pltpu.VMEM((2 * F2,), jnp.float32),
            pltpu.VMEM((CH,), jnp.int32),
            pltpu.VMEM((CH,), jnp.int32),
            pltpu.VMEM((CH, F2), jnp.float32),
            pltpu.VMEM((CH, F2), jnp.float32),
            pltpu.VMEM((SEG_T * F2,), jnp.float32),
            pltpu.SemaphoreType.DMA,
            pltpu.SemaphoreType.DMA,
        ],
    )(h2p, idx_pad, splits, aff)
    return out.reshape(N_SEG_PAD, F2)[:N_SEG]


# speedup vs baseline: 2.7412x; 1.1257x over previous
import jax
import jax.numpy as jnp
from jax import lax
from jax.experimental import pallas as pl
from jax.experimental.pallas import tpu as pltpu
from jax.experimental.pallas import tpu_sc as plsc

N_POINTS = 1600000
N_SEG = 100000
F1 = 9
F2 = 32
EPS = 1e-5

BLK = 8000
GRID = N_POINTS // BLK
CH = 112
N_PAD = N_POINTS + CH
N_TILES = 32
SEG_T = 3128
N_SEG_PAD = N_TILES * SEG_T


def _stats1_body(x_ref, w1t_ref, b1_ref, out_ref):
    i = pl.program_id(0)
    h = jnp.dot(x_ref[...], w1t_ref[...], preferred_element_type=jnp.float32)
    h = h + b1_ref[...]
    s = jnp.sum(h, axis=0, keepdims=True)
    ss = jnp.sum(h * h, axis=0, keepdims=True)
    blk = jnp.concatenate([s, ss], axis=0)

    @pl.when(i == 0)
    def _():
        out_ref[...] = blk

    @pl.when(i != 0)
    def _():
        out_ref[...] = out_ref[...] + blk


def _fwd_body(x_ref, w1t_ref, b1_ref, sc1_ref, sh1_ref, w2t_ref, b2_ref,
              h2_ref, st_ref):
    i = pl.program_id(0)
    h1 = jnp.dot(x_ref[...], w1t_ref[...], preferred_element_type=jnp.float32)
    h1 = h1 + b1_ref[...]
    f1 = jnp.maximum(h1 * sc1_ref[...] + sh1_ref[...], 0.0)
    h2 = jnp.dot(f1, w2t_ref[...], preferred_element_type=jnp.float32)
    h2 = h2 + b2_ref[...]
    h2_ref[...] = h2
    s = jnp.sum(h2, axis=0, keepdims=True)
    ss = jnp.sum(h2 * h2, axis=0, keepdims=True)
    blk = jnp.concatenate([s, ss], axis=0)

    @pl.when(i == 0)
    def _():
        st_ref[...] = blk

    @pl.when(i != 0)
    def _():
        st_ref[...] = st_ref[...] + blk


def _segmax_body(h2_hbm, idx_hbm, splits_hbm, aff_hbm, out_hbm,
                 splits_v, aff_v, idxbuf0, idxbuf1, rowbuf0, rowbuf1,
                 slab, sem0, sem1):
    cid = lax.axis_index("c")
    sid = lax.axis_index("s")
    wid = sid * 2 + cid
    pltpu.sync_copy(splits_hbm, splits_v)
    pltpu.sync_copy(aff_hbm, aff_v)
    seg_lo = pl.multiple_of(wid * SEG_T, 8)
    rvec = splits_v[pl.ds(wid, 16)]
    r0 = rvec[0]
    r1 = rvec[1]
    r0a = jnp.bitwise_and(r0, jnp.int32(-8))
    nchunks = (r1 - r0a + (CH - 1)) // CH

    def start(c, ib, rb, sem):
        base = pl.multiple_of(r0a + c * CH, 8)
        pltpu.async_copy(idx_hbm.at[pl.ds(base, CH)], ib, sem)
        pltpu.async_copy(h2_hbm.at[pl.ds(base, CH), :], rb, sem)

    def drain(ib, rb, sem):
        pltpu.make_async_copy(idx_hbm.at[pl.ds(0, CH)], ib, sem).wait()
        pltpu.make_async_copy(h2_hbm.at[pl.ds(0, CH), :], rb, sem).wait()

    @pl.when(nchunks > 0)
    def _():
        start(0, idxbuf0, rowbuf0, sem0)

    @pl.when(nchunks > 1)
    def _():
        start(1, idxbuf1, rowbuf1, sem1)

    ninf = jnp.full((16,), -jnp.inf, dtype=jnp.float32)

    def init_body(s, carry):
        slab[pl.ds(s * F2, 16)] = ninf
        slab[pl.ds(s * F2 + 16, 16)] = ninf
        return carry

    lax.fori_loop(0, SEG_T, init_body, 0)

    def do_row(local, rb, r):
        @pl.when(jnp.logical_and(local >= 0, local < SEG_T))
        def _():
            o = local * F2
            slab[pl.ds(o, 16)] = jnp.maximum(slab[pl.ds(o, 16)],
                                             rb[r, 0:16])
            slab[pl.ds(o + 16, 16)] = jnp.maximum(slab[pl.ds(o + 16, 16)],
                                                  rb[r, 16:32])

    def process(ib, rb):
        def group_body(g, carry):
            iv = ib[pl.ds(g * 16, 16)]
            for j in range(16):
                do_row(iv[j] - seg_lo, rb, g * 16 + j)
            return carry
        lax.fori_loop(0, CH // 16, group_body, 0)

    def pair_body(k, carry):
        c0 = 2 * k
        c1 = c0 + 1

        @pl.when(c0 < nchunks)
        def _():
            drain(idxbuf0, rowbuf0, sem0)
            process(idxbuf0, rowbuf0)

            @pl.when(c0 + 2 < nchunks)
            def _():
                start(c0 + 2, idxbuf0, rowbuf0, sem0)

        @pl.when(c1 < nchunks)
        def _():
            drain(idxbuf1, rowbuf1, sem1)
            process(idxbuf1, rowbuf1)

            @pl.when(c1 + 2 < nchunks)
            def _():
                start(c1 + 2, idxbuf1, rowbuf1, sem1)

        return carry

    lax.fori_loop(0, (nchunks + 1) // 2, pair_body, 0)

    sca = aff_v[pl.ds(0, 16)]
    scb = aff_v[pl.ds(16, 16)]
    sha = aff_v[pl.ds(32, 16)]
    shb = aff_v[pl.ds(48, 16)]
    zero = jnp.zeros((16,), dtype=jnp.float32)

    def fin_body(s, carry):
        o = s * F2
        slab[pl.ds(o, 16)] = jnp.maximum(slab[pl.ds(o, 16)] * sca + sha, zero)
        slab[pl.ds(o + 16, 16)] = jnp.maximum(
            slab[pl.ds(o + 16, 16)] * scb + shb, zero)
        return carry

    lax.fori_loop(0, SEG_T, fin_body, 0)
    out_lo = pl.multiple_of(wid * (SEG_T * F2), 8)
    pltpu.sync_copy(slab, out_hbm.at[pl.ds(out_lo, SEG_T * F2)])


def kernel(points, inverse_indices, W1, b1, g1, be1, W2, b2, g2, be2):
    idx = inverse_indices.astype(jnp.int32)
    w1t = W1.T
    w2t = W2.T
    b1r = b1.reshape(1, F2)

    st1 = pl.pallas_call(
        _stats1_body,
        grid=(GRID,),
        in_specs=[
            pl.BlockSpec((BLK, F1), lambda i: (i, 0)),
            pl.BlockSpec((F1, F2), lambda i: (0, 0)),
            pl.BlockSpec((1, F2), lambda i: (0, 0)),
        ],
        out_specs=pl.BlockSpec((2, F2), lambda i: (0, 0)),
        out_shape=jax.ShapeDtypeStruct((2, F2), jnp.float32),
        compiler_params=pltpu.CompilerParams(
            dimension_semantics=("arbitrary",)),
    )(points, w1t, b1r)

    inv_n = jnp.float32(1.0 / N_POINTS)
    mean1 = st1[0] * inv_n
    var1 = st1[1] * inv_n - mean1 * mean1
    sc1 = g1 / jnp.sqrt(var1 + EPS)
    sh1 = be1 - mean1 * sc1

    h2p, st2 = pl.pallas_call(
        _fwd_body,
        grid=(GRID,),
        in_specs=[
            pl.BlockSpec((BLK, F1), lambda i: (i, 0)),
            pl.BlockSpec((F1, F2), lambda i: (0, 0)),
            pl.BlockSpec((1, F2), lambda i: (0, 0)),
            pl.BlockSpec((1, F2), lambda i: (0, 0)),
            pl.BlockSpec((1, F2), lambda i: (0, 0)),
            pl.BlockSpec((F2, F2), lambda i: (0, 0)),
            pl.BlockSpec((1, F2), lambda i: (0, 0)),
        ],
        out_specs=[
            pl.BlockSpec((BLK, F2), lambda i: (i, 0)),
            pl.BlockSpec((2, F2), lambda i: (0, 0)),
        ],
        out_shape=[
            jax.ShapeDtypeStruct((N_PAD, F2), jnp.float32),
            jax.ShapeDtypeStruct((2, F2), jnp.float32),
        ],
        compiler_params=pltpu.CompilerParams(
            dimension_semantics=("arbitrary",)),
    )(points, w1t, b1r, sc1.reshape(1, F2), sh1.reshape(1, F2), w2t,
      b2.reshape(1, F2))

    mean2 = st2[0] * inv_n
    var2 = st2[1] * inv_n - mean2 * mean2
    sc2 = g2 / jnp.sqrt(var2 + EPS)
    sh2 = be2 - mean2 * sc2
    aff = jnp.concatenate([sc2, sh2]).astype(jnp.float32)

    idx_pad = jnp.concatenate(
        [idx, jnp.full((CH,), jnp.int32(2**31 - 2))])
    bounds = jnp.arange(0, N_SEG_PAD + 1, SEG_T, dtype=jnp.int32)
    splits = jnp.searchsorted(idx, bounds).astype(jnp.int32)
    splits = jnp.concatenate([splits, jnp.zeros((15,), jnp.int32)])

    mesh = plsc.VectorSubcoreMesh(core_axis_name="c", subcore_axis_name="s")
    out = pl.kernel(
        _segmax_body,
        mesh=mesh,
        out_type=jax.ShapeDtypeStruct((N_SEG_PAD * F2,), jnp.float32),
        scratch_types=[
            pltpu.VMEM((48,), jnp.int32),
            pltpu.VMEM((2 * F2,), jnp.float32),
            pltpu.VMEM((CH,), jnp.int32),
            pltpu.VMEM((CH,), jnp.int32),
            pltpu.VMEM((CH, F2), jnp.float32),
            pltpu.VMEM((CH, F2), jnp.float32),
            pltpu.VMEM((SEG_T * F2,), jnp.float32),
            pltpu.SemaphoreType.DMA,
            pltpu.SemaphoreType.DMA,
        ],
    )(h2p, idx_pad, splits, aff)
    return out.reshape(N_SEG_PAD, F2)[:N_SEG]


# speedup vs baseline: 2.8805x; 1.0508x over previous
import jax
import jax.numpy as jnp
from jax import lax
from jax.experimental import pallas as pl
from jax.experimental.pallas import tpu as pltpu
from jax.experimental.pallas import tpu_sc as plsc

N_POINTS = 1600000
N_SEG = 100000
F1 = 9
F2 = 32
EPS = 1e-5

BLK = 16000
GRID = N_POINTS // BLK
CH = 112
N_PAD = N_POINTS + CH
N_TILES = 32
SEG_T = 3128
N_SEG_PAD = N_TILES * SEG_T


def _stats1_body(x_ref, w1t_ref, b1_ref, out_ref):
    i = pl.program_id(0)
    h = jnp.dot(x_ref[...], w1t_ref[...], preferred_element_type=jnp.float32)
    h = h + b1_ref[...]
    s = jnp.sum(h, axis=0, keepdims=True)
    ss = jnp.sum(h * h, axis=0, keepdims=True)
    blk = jnp.concatenate([s, ss], axis=0)

    @pl.when(i == 0)
    def _():
        out_ref[...] = blk

    @pl.when(i != 0)
    def _():
        out_ref[...] = out_ref[...] + blk


def _fwd_body(x_ref, w1t_ref, b1_ref, sc1_ref, sh1_ref, w2t_ref, b2_ref,
              h2_ref, st_ref):
    i = pl.program_id(0)
    h1 = jnp.dot(x_ref[...], w1t_ref[...], preferred_element_type=jnp.float32)
    h1 = h1 + b1_ref[...]
    f1 = jnp.maximum(h1 * sc1_ref[...] + sh1_ref[...], 0.0)
    h2 = jnp.dot(f1, w2t_ref[...], preferred_element_type=jnp.float32)
    h2 = h2 + b2_ref[...]
    h2_ref[...] = h2
    s = jnp.sum(h2, axis=0, keepdims=True)
    ss = jnp.sum(h2 * h2, axis=0, keepdims=True)
    blk = jnp.concatenate([s, ss], axis=0)

    @pl.when(i == 0)
    def _():
        st_ref[...] = blk

    @pl.when(i != 0)
    def _():
        st_ref[...] = st_ref[...] + blk


def _segmax_body(h2_hbm, idx_hbm, splits_hbm, aff_hbm, out_hbm,
                 splits_v, aff_v, idxbuf0, idxbuf1, rowbuf0, rowbuf1,
                 slab, sem0, sem1):
    cid = lax.axis_index("c")
    sid = lax.axis_index("s")
    wid = sid * 2 + cid
    pltpu.sync_copy(splits_hbm, splits_v)
    pltpu.sync_copy(aff_hbm, aff_v)
    seg_lo = pl.multiple_of(wid * SEG_T, 8)
    rvec = splits_v[pl.ds(wid, 16)]
    r0 = rvec[0]
    r1 = rvec[1]
    r0a = jnp.bitwise_and(r0, jnp.int32(-8))
    nchunks = (r1 - r0a + (CH - 1)) // CH

    def start(c, ib, rb, sem):
        base = pl.multiple_of(r0a + c * CH, 8)
        pltpu.async_copy(idx_hbm.at[pl.ds(base, CH)], ib, sem)
        pltpu.async_copy(h2_hbm.at[pl.ds(base, CH), :], rb, sem)

    def drain(ib, rb, sem):
        pltpu.make_async_copy(idx_hbm.at[pl.ds(0, CH)], ib, sem).wait()
        pltpu.make_async_copy(h2_hbm.at[pl.ds(0, CH), :], rb, sem).wait()

    @pl.when(nchunks > 0)
    def _():
        start(0, idxbuf0, rowbuf0, sem0)

    @pl.when(nchunks > 1)
    def _():
        start(1, idxbuf1, rowbuf1, sem1)

    ninf = jnp.full((16,), -jnp.inf, dtype=jnp.float32)

    def init_body(s, carry):
        slab[pl.ds(s * F2, 16)] = ninf
        slab[pl.ds(s * F2 + 16, 16)] = ninf
        return carry

    lax.fori_loop(0, SEG_T, init_body, 0)

    def do_row(local, rb, r):
        @pl.when(jnp.logical_and(local >= 0, local < SEG_T))
        def _():
            o = local * F2
            slab[pl.ds(o, 16)] = jnp.maximum(slab[pl.ds(o, 16)],
                                             rb[r, 0:16])
            slab[pl.ds(o + 16, 16)] = jnp.maximum(slab[pl.ds(o + 16, 16)],
                                                  rb[r, 16:32])

    def process(ib, rb):
        def group_body(g, carry):
            iv = ib[pl.ds(g * 16, 16)]
            for j in range(16):
                do_row(iv[j] - seg_lo, rb, g * 16 + j)
            return carry
        lax.fori_loop(0, CH // 16, group_body, 0)

    def pair_body(k, carry):
        c0 = 2 * k
        c1 = c0 + 1

        @pl.when(c0 < nchunks)
        def _():
            drain(idxbuf0, rowbuf0, sem0)
            process(idxbuf0, rowbuf0)

            @pl.when(c0 + 2 < nchunks)
            def _():
                start(c0 + 2, idxbuf0, rowbuf0, sem0)

        @pl.when(c1 < nchunks)
        def _():
            drain(idxbuf1, rowbuf1, sem1)
            process(idxbuf1, rowbuf1)

            @pl.when(c1 + 2 < nchunks)
            def _():
                start(c1 + 2, idxbuf1, rowbuf1, sem1)

        return carry

    lax.fori_loop(0, (nchunks + 1) // 2, pair_body, 0)

    sca = aff_v[pl.ds(0, 16)]
    scb = aff_v[pl.ds(16, 16)]
    sha = aff_v[pl.ds(32, 16)]
    shb = aff_v[pl.ds(48, 16)]
    zero = jnp.zeros((16,), dtype=jnp.float32)

    def fin_body(s, carry):
        o = s * F2
        slab[pl.ds(o, 16)] = jnp.maximum(slab[pl.ds(o, 16)] * sca + sha, zero)
        slab[pl.ds(o + 16, 16)] = jnp.maximum(
            slab[pl.ds(o + 16, 16)] * scb + shb, zero)
        return carry

    lax.fori_loop(0, SEG_T, fin_body, 0)
    out_lo = pl.multiple_of(wid * (SEG_T * F2), 8)
    pltpu.sync_copy(slab, out_hbm.at[pl.ds(out_lo, SEG_T * F2)])


def kernel(points, inverse_indices, W1, b1, g1, be1, W2, b2, g2, be2):
    idx = inverse_indices.astype(jnp.int32)
    w1t = W1.T
    w2t = W2.T
    b1r = b1.reshape(1, F2)

    st1 = pl.pallas_call(
        _stats1_body,
        grid=(GRID,),
        in_specs=[
            pl.BlockSpec((BLK, F1), lambda i: (i, 0)),
            pl.BlockSpec((F1, F2), lambda i: (0, 0)),
            pl.BlockSpec((1, F2), lambda i: (0, 0)),
        ],
        out_specs=pl.BlockSpec((2, F2), lambda i: (0, 0)),
        out_shape=jax.ShapeDtypeStruct((2, F2), jnp.float32),
        compiler_params=pltpu.CompilerParams(
            dimension_semantics=("arbitrary",)),
    )(points, w1t, b1r)

    inv_n = jnp.float32(1.0 / N_POINTS)
    mean1 = st1[0] * inv_n
    var1 = st1[1] * inv_n - mean1 * mean1
    sc1 = g1 / jnp.sqrt(var1 + EPS)
    sh1 = be1 - mean1 * sc1

    h2p, st2 = pl.pallas_call(
        _fwd_body,
        grid=(GRID,),
        in_specs=[
            pl.BlockSpec((BLK, F1), lambda i: (i, 0)),
            pl.BlockSpec((F1, F2), lambda i: (0, 0)),
            pl.BlockSpec((1, F2), lambda i: (0, 0)),
            pl.BlockSpec((1, F2), lambda i: (0, 0)),
            pl.BlockSpec((1, F2), lambda i: (0, 0)),
            pl.BlockSpec((F2, F2), lambda i: (0, 0)),
            pl.BlockSpec((1, F2), lambda i: (0, 0)),
        ],
        out_specs=[
            pl.BlockSpec((BLK, F2), lambda i: (i, 0)),
            pl.BlockSpec((2, F2), lambda i: (0, 0)),
        ],
        out_shape=[
            jax.ShapeDtypeStruct((N_PAD, F2), jnp.float32),
            jax.ShapeDtypeStruct((2, F2), jnp.float32),
        ],
        compiler_params=pltpu.CompilerParams(
            dimension_semantics=("arbitrary",)),
    )(points, w1t, b1r, sc1.reshape(1, F2), sh1.reshape(1, F2), w2t,
      b2.reshape(1, F2))

    mean2 = st2[0] * inv_n
    var2 = st2[1] * inv_n - mean2 * mean2
    sc2 = g2 / jnp.sqrt(var2 + EPS)
    sh2 = be2 - mean2 * sc2
    aff = jnp.concatenate([sc2, sh2]).astype(jnp.float32)

    idx_pad = jnp.concatenate(
        [idx, jnp.full((CH,), jnp.int32(2**31 - 2))])
    bounds = jnp.arange(0, N_SEG_PAD + 1, SEG_T, dtype=jnp.int32)
    splits = jnp.searchsorted(idx, bounds).astype(jnp.int32)
    splits = jnp.concatenate([splits, jnp.zeros((15,), jnp.int32)])

    mesh = plsc.VectorSubcoreMesh(core_axis_name="c", subcore_axis_name="s")
    out = pl.kernel(
        _segmax_body,
        mesh=mesh,
        out_type=jax.ShapeDtypeStruct((N_SEG_PAD * F2,), jnp.float32),
        scratch_types=[
            pltpu.VMEM((48,), jnp.int32),
            pltpu.VMEM((2 * F2,), jnp.float32),
            pltpu.VMEM((CH,), jnp.int32),
            pltpu.VMEM((CH,), jnp.int32),
            pltpu.VMEM((CH, F2), jnp.float32),
            pltpu.VMEM((CH, F2), jnp.float32),
            pltpu.VMEM((SEG_T * F2,), jnp.float32),
            pltpu.SemaphoreType.DMA,
            pltpu.SemaphoreType.DMA,
        ],
    )(h2p, idx_pad, splits, aff)
    return out.reshape(N_SEG_PAD, F2)[:N_SEG]
